# baseline (speedup 1.0000x reference)
import functools
import math

import jax
import jax.numpy as jnp
from jax import lax
from jax.experimental import pallas as pl
from jax.experimental.pallas import tpu as pltpu
from jax.experimental.pallas import tpu_sc as plsc

N = 2048
D = 1024
H = 1024
E = 8
K = 2
EPS = 1e-5
_INV_SQRT2 = 1.0 / math.sqrt(2.0)

T = 128
NT = (N * K) // T + E
PADTOT = NT * T
NPAIR = N * K
CB = 256

NC, NS = 2, 16
NW = NC * NS
ROW_W = PADTOT // NW
CH = 80
TOK_W = N // NW
TSUB = 32


def _gate_route_body(x_ref, wg_ref, bg_ref,
                     pos0_ref, pos1_ref, w0_ref, w1_ref, te_ref, tv_ref):
    x = x_ref[...]
    logits = jnp.dot(x, wg_ref[...],
                     preferred_element_type=jnp.float32) + bg_ref[0]
    m = jnp.max(logits, axis=-1, keepdims=True)
    ex = jnp.exp(logits - m)
    probs = ex / jnp.sum(ex, axis=-1, keepdims=True)
    cols8 = lax.broadcasted_iota(jnp.int32, (1, E), 1)
    masks = []
    for j in range(E):
        pj = probs[:, j:j + 1]
        rk = (jnp.sum((probs > pj).astype(jnp.float32), axis=1,
                      keepdims=True)
              + jnp.sum(((probs == pj) & (cols8 < j)).astype(jnp.float32),
                        axis=1, keepdims=True))
        masks.append((rk < float(K)).astype(jnp.float32))
    mask = jnp.concatenate(masks, axis=1)
    wsel = probs * mask
    wn = wsel / jnp.sum(wsel, axis=1, keepdims=True)

    ri = lax.broadcasted_iota(jnp.int32, (CB, CB), 0)
    ci = lax.broadcasted_iota(jnp.int32, (CB, CB), 1)
    ltri = (ci < ri).astype(jnp.bfloat16)
    prev = jnp.zeros((1, E), jnp.float32)
    ranks = []
    for b in range(N // CB):
        mb = mask[b * CB:(b + 1) * CB, :]
        ranks.append(jnp.dot(ltri, mb.astype(jnp.bfloat16),
                             preferred_element_type=jnp.float32) + prev)
        prev = prev + jnp.sum(mb, axis=0, keepdims=True)
    rank = jnp.concatenate(ranks, axis=0)
    counts = prev

    ntile = jnp.floor((counts + float(T - 1)) * (1.0 / T))
    ut = (lax.broadcasted_iota(jnp.int32, (E, E), 0)
          <= lax.broadcasted_iota(jnp.int32, (E, E), 1)).astype(jnp.float32)
    cum = jnp.dot(ntile, ut, preferred_element_type=jnp.float32)
    row_off = (cum - ntile) * float(T)
    pos = row_off + rank

    eidx = lax.broadcasted_iota(jnp.int32, (N, E), 1).astype(jnp.float32)
    e0 = jnp.min(jnp.where(mask > 0, eidx, 1e9), axis=1, keepdims=True)
    e1 = jnp.max(jnp.where(mask > 0, eidx, -1.0), axis=1, keepdims=True)
    oh0 = (eidx == e0).astype(jnp.float32)
    oh1 = (eidx == e1).astype(jnp.float32)
    pos0_ref[...] = jnp.sum(pos * oh0, axis=1,
                            keepdims=True).astype(jnp.int32)
    pos1_ref[...] = jnp.sum(pos * oh1, axis=1,
                            keepdims=True).astype(jnp.int32)
    w0_ref[...] = jnp.sum(wn * oh0, axis=1, keepdims=True)
    w1_ref[...] = jnp.sum(wn * oh1, axis=1, keepdims=True)

    ti = lax.broadcasted_iota(jnp.int32, (1, NT), 1).astype(jnp.float32)
    te = jnp.zeros((1, NT), jnp.float32)
    for e in range(E):
        te = te + (ti >= cum[:, e:e + 1]).astype(jnp.float32)
    te_ref[...] = jnp.minimum(te, float(E - 1)).astype(jnp.int32)
    tv_ref[...] = (ti < cum[:, E - 1:E]).astype(jnp.int32)


def _gate_route(x, Wg, bg2):
    return pl.pallas_call(
        _gate_route_body,
        out_shape=(
            jax.ShapeDtypeStruct((N, 1), jnp.int32),
            jax.ShapeDtypeStruct((N, 1), jnp.int32),
            jax.ShapeDtypeStruct((N, 1), jnp.float32),
            jax.ShapeDtypeStruct((N, 1), jnp.float32),
            jax.ShapeDtypeStruct((1, NT), jnp.int32),
            jax.ShapeDtypeStruct((1, NT), jnp.int32),
        ),
    )(x, Wg, bg2)


def _route_gather_body(pos_hbm, w_hbm, x_hbm, xg_hbm, ws_hbm,
                       pos_v, wv, tok_s, ws_s, rows_v, sem):
    wid = lax.axis_index("s") * NC + lax.axis_index("c")
    pltpu.sync_copy(pos_hbm, pos_v)
    pltpu.sync_copy(w_hbm, wv)

    zi = jnp.zeros((16,), jnp.int32)
    zf = jnp.zeros((16,), jnp.float32)

    def init_body(i, _):
        tok_s[pl.ds(i * 16, 16)] = zi
        ws_s[pl.ds(i * 16, 16)] = zf
        return 0
    lax.fori_loop(0, PADTOT // 16, init_body, 0)

    lane = lax.iota(jnp.int32, 16)

    def scat_body(i, _):
        base = i * 16
        jv = base + lane
        tok = jnp.where(jv < N, jv, jv - N)
        idx = pos_v[pl.ds(base, 16)]
        plsc.store_scatter(tok_s, [idx], tok)
        plsc.store_scatter(ws_s, [idx], wv[pl.ds(base, 16)])
        return 0
    lax.fori_loop(0, NPAIR // 16, scat_body, 0)

    base = wid * ROW_W
    for c in range(ROW_W // CH):
        cb = base + c * CH
        pltpu.async_copy(x_hbm.at[tok_s.at[pl.ds(cb, CH)]], rows_v,
                         sem).wait()
        pltpu.sync_copy(rows_v, xg_hbm.at[pl.ds(cb, CH)])
    pltpu.sync_copy(ws_s.at[pl.ds(base, ROW_W)],
                    ws_hbm.at[pl.ds(base, ROW_W)])


def _route_gather(posf, wf, x):
    mesh = plsc.VectorSubcoreMesh(core_axis_name="c", subcore_axis_name="s")
    return pl.kernel(
        _route_gather_body,
        out_type=(
            jax.ShapeDtypeStruct((PADTOT, D), jnp.float32),
            jax.ShapeDtypeStruct((PADTOT,), jnp.float32),
        ),
        mesh=mesh,
        compiler_params=pltpu.CompilerParams(needs_layout_passes=False),
        scratch_types=[
            pltpu.VMEM((NPAIR,), jnp.int32),
            pltpu.VMEM((NPAIR,), jnp.float32),
            pltpu.VMEM((PADTOT,), jnp.int32),
            pltpu.VMEM((PADTOT,), jnp.float32),
            pltpu.VMEM((CH, D), jnp.float32),
            pltpu.SemaphoreType.DMA,
        ],
    )(posf, wf, x)


def _expert_body(te_ref, tv_ref, xg_ref, ws_ref,
                 w1_ref, b1_ref, g1_ref, be1_ref, w2_ref, b2_ref, eo_ref):
    t = pl.program_id(0)

    @pl.when(tv_ref[t] == 1)
    def _():
        xb = xg_ref[...].astype(jnp.bfloat16)
        h = jnp.dot(xb, w1_ref[0].astype(jnp.bfloat16),
                    preferred_element_type=jnp.float32) + b1_ref[0]
        mu = jnp.mean(h, axis=-1, keepdims=True)
        var = jnp.mean((h - mu) * (h - mu), axis=-1, keepdims=True)
        hn = (h - mu) * lax.rsqrt(var + EPS) * g1_ref[0] + be1_ref[0]
        ha = hn * 0.5 * (1.0 + lax.erf(hn * _INV_SQRT2))
        eo = jnp.dot(ha.astype(jnp.bfloat16), w2_ref[0].astype(jnp.bfloat16),
                     preferred_element_type=jnp.float32) + b2_ref[0]
        wsc = ws_ref[...]
        eo_ref[...] = jnp.where(wsc > 0.0, eo * wsc, 0.0)


def _expert_ffn(ten, tvn, xg, ws2, W1, b1r, g1r, be1r, W2, b2r):
    grid_spec = pltpu.PrefetchScalarGridSpec(
        num_scalar_prefetch=2,
        grid=(NT,),
        in_specs=[
            pl.BlockSpec((T, D), lambda t, te, tv: (t, 0)),
            pl.BlockSpec((T, 1), lambda t, te, tv: (t, 0)),
            pl.BlockSpec((1, D, H), lambda t, te, tv: (te[t], 0, 0)),
            pl.BlockSpec((1, 1, H), lambda t, te, tv: (te[t], 0, 0)),
            pl.BlockSpec((1, 1, H), lambda t, te, tv: (te[t], 0, 0)),
            pl.BlockSpec((1, 1, H), lambda t, te, tv: (te[t], 0, 0)),
            pl.BlockSpec((1, H, H), lambda t, te, tv: (te[t], 0, 0)),
            pl.BlockSpec((1, 1, H), lambda t, te, tv: (te[t], 0, 0)),
        ],
        out_specs=pl.BlockSpec((T, H), lambda t, te, tv: (t, 0)),
    )
    return pl.pallas_call(
        _expert_body,
        grid_spec=grid_spec,
        out_shape=jax.ShapeDtypeStruct((PADTOT, H), jnp.float32),
    )(ten, tvn, xg, ws2, W1, b1r, g1r, be1r, W2, b2r)


def _combine_body(eo_hbm, pos_hbm, out_hbm,
                  p0_v, p1_v, rows0_v, rows1_v, acc_v, sem):
    wid = lax.axis_index("s") * NC + lax.axis_index("c")
    tokbase = wid * TOK_W
    for sc in range(TOK_W // TSUB):
        tb = tokbase + sc * TSUB
        pltpu.sync_copy(pos_hbm.at[pl.ds(tb, TSUB)], p0_v)
        pltpu.sync_copy(pos_hbm.at[pl.ds(N + tb, TSUB)], p1_v)
        pltpu.async_copy(eo_hbm.at[p0_v], rows0_v, sem).wait()
        pltpu.async_copy(eo_hbm.at[p1_v], rows1_v, sem).wait()

        def add_body(i, _):
            t = i // (H // 16)
            c = i % (H // 16)
            sl = pl.ds(c * 16, 16)
            acc_v[t, sl] = rows0_v[t, sl] + rows1_v[t, sl]
            return 0
        lax.fori_loop(0, TSUB * (H // 16), add_body, 0)
        pltpu.sync_copy(acc_v, out_hbm.at[pl.ds(tb, TSUB)])


def _combine(eoF, posf):
    mesh = plsc.VectorSubcoreMesh(core_axis_name="c", subcore_axis_name="s")
    return pl.kernel(
        _combine_body,
        out_type=jax.ShapeDtypeStruct((N, H), jnp.float32),
        mesh=mesh,
        compiler_params=pltpu.CompilerParams(needs_layout_passes=False),
        scratch_types=[
            pltpu.VMEM((TSUB,), jnp.int32),
            pltpu.VMEM((TSUB,), jnp.int32),
            pltpu.VMEM((TSUB, H), jnp.float32),
            pltpu.VMEM((TSUB, H), jnp.float32),
            pltpu.VMEM((TSUB, H), jnp.float32),
            pltpu.SemaphoreType.DMA,
        ],
    )(eoF, posf)


@jax.jit
def kernel(x, W1, b1, g1, be1, W2, b2, Wg, bg):
    bg2 = bg.reshape(1, E)
    b1r = b1.reshape(E, 1, H)
    g1r = g1.reshape(E, 1, H)
    be1r = be1.reshape(E, 1, H)
    b2r = b2.reshape(E, 1, H)

    pos0, pos1, w0, w1, te, tv = _gate_route(x, Wg, bg2)
    posf = jnp.concatenate([pos0.reshape(N), pos1.reshape(N)])
    wf = jnp.concatenate([w0.reshape(N), w1.reshape(N)])
    ten = te.reshape(NT)
    tvn = tv.reshape(NT)

    xg, ws = _route_gather(posf, wf, x)
    eoF = _expert_ffn(ten, tvn, xg, ws.reshape(PADTOT, 1),
                      W1, b1r, g1r, be1r, W2, b2r)
    out = _combine(eoF, posf)
    return out


# speedup vs baseline: 1.0740x; 1.0740x over previous
import functools
import math

import jax
import jax.numpy as jnp
from jax import lax
from jax.experimental import pallas as pl
from jax.experimental.pallas import tpu as pltpu
from jax.experimental.pallas import tpu_sc as plsc

N = 2048
D = 1024
H = 1024
E = 8
K = 2
EPS = 1e-5
_INV_SQRT2 = 1.0 / math.sqrt(2.0)

T = 128
NT = (N * K) // T + E
PADTOT = NT * T
NPAIR = N * K
CB = 256

NC, NS = 2, 16
NW = NC * NS
ROW_W = PADTOT // NW
CH = 40
TOK_W = N // NW
TSUB = 32


def _gate_route_body(x_ref, wg_ref, bg_ref,
                     pos0_ref, pos1_ref, w0_ref, w1_ref, te_ref, tv_ref):
    x = x_ref[...]
    logits = jnp.dot(x, wg_ref[...],
                     preferred_element_type=jnp.float32) + bg_ref[0]
    m = jnp.max(logits, axis=-1, keepdims=True)
    ex = jnp.exp(logits - m)
    probs = ex / jnp.sum(ex, axis=-1, keepdims=True)
    cols8 = lax.broadcasted_iota(jnp.int32, (1, E), 1)
    masks = []
    for j in range(E):
        pj = probs[:, j:j + 1]
        rk = (jnp.sum((probs > pj).astype(jnp.float32), axis=1,
                      keepdims=True)
              + jnp.sum(((probs == pj) & (cols8 < j)).astype(jnp.float32),
                        axis=1, keepdims=True))
        masks.append((rk < float(K)).astype(jnp.float32))
    mask = jnp.concatenate(masks, axis=1)
    wsel = probs * mask
    wn = wsel / jnp.sum(wsel, axis=1, keepdims=True)

    ri = lax.broadcasted_iota(jnp.int32, (CB, CB), 0)
    ci = lax.broadcasted_iota(jnp.int32, (CB, CB), 1)
    ltri = (ci < ri).astype(jnp.bfloat16)
    prev = jnp.zeros((1, E), jnp.float32)
    ranks = []
    for b in range(N // CB):
        mb = mask[b * CB:(b + 1) * CB, :]
        ranks.append(jnp.dot(ltri, mb.astype(jnp.bfloat16),
                             preferred_element_type=jnp.float32) + prev)
        prev = prev + jnp.sum(mb, axis=0, keepdims=True)
    rank = jnp.concatenate(ranks, axis=0)
    counts = prev

    ntile = jnp.floor((counts + float(T - 1)) * (1.0 / T))
    ut = (lax.broadcasted_iota(jnp.int32, (E, E), 0)
          <= lax.broadcasted_iota(jnp.int32, (E, E), 1)).astype(jnp.float32)
    cum = jnp.dot(ntile, ut, preferred_element_type=jnp.float32)
    row_off = (cum - ntile) * float(T)
    pos = row_off + rank

    eidx = lax.broadcasted_iota(jnp.int32, (N, E), 1).astype(jnp.float32)
    e0 = jnp.min(jnp.where(mask > 0, eidx, 1e9), axis=1, keepdims=True)
    e1 = jnp.max(jnp.where(mask > 0, eidx, -1.0), axis=1, keepdims=True)
    oh0 = (eidx == e0).astype(jnp.float32)
    oh1 = (eidx == e1).astype(jnp.float32)
    pos0_ref[...] = jnp.sum(pos * oh0, axis=1,
                            keepdims=True).astype(jnp.int32)
    pos1_ref[...] = jnp.sum(pos * oh1, axis=1,
                            keepdims=True).astype(jnp.int32)
    w0_ref[...] = jnp.sum(wn * oh0, axis=1, keepdims=True)
    w1_ref[...] = jnp.sum(wn * oh1, axis=1, keepdims=True)

    ti = lax.broadcasted_iota(jnp.int32, (1, NT), 1).astype(jnp.float32)
    te = jnp.zeros((1, NT), jnp.float32)
    for e in range(E):
        te = te + (ti >= cum[:, e:e + 1]).astype(jnp.float32)
    te_ref[...] = jnp.minimum(te, float(E - 1)).astype(jnp.int32)
    tv_ref[...] = (ti < cum[:, E - 1:E]).astype(jnp.int32)


def _gate_route(x, Wg, bg2):
    return pl.pallas_call(
        _gate_route_body,
        out_shape=(
            jax.ShapeDtypeStruct((N, 1), jnp.int32),
            jax.ShapeDtypeStruct((N, 1), jnp.int32),
            jax.ShapeDtypeStruct((N, 1), jnp.float32),
            jax.ShapeDtypeStruct((N, 1), jnp.float32),
            jax.ShapeDtypeStruct((1, NT), jnp.int32),
            jax.ShapeDtypeStruct((1, NT), jnp.int32),
        ),
    )(x, Wg, bg2)


def _route_gather_body(pos_hbm, w_hbm, x_hbm, xg_hbm, ws_hbm,
                       pos_v, wv, tok_s, ws_s, rows_v, rows2_v, sem, sem2):
    wid = lax.axis_index("s") * NC + lax.axis_index("c")
    pltpu.sync_copy(pos_hbm, pos_v)
    pltpu.sync_copy(w_hbm, wv)

    zi = jnp.zeros((16,), jnp.int32)
    zf = jnp.zeros((16,), jnp.float32)

    @plsc.parallel_loop(0, PADTOT // 16, unroll=4)
    def _init(i):
        tok_s[pl.ds(i * 16, 16)] = zi
        ws_s[pl.ds(i * 16, 16)] = zf

    lane = lax.iota(jnp.int32, 16)

    @plsc.parallel_loop(0, NPAIR // 16, unroll=4)
    def _scat(i):
        base = i * 16
        jv = base + lane
        tok = jnp.where(jv < N, jv, jv - N)
        idx = pos_v[pl.ds(base, 16)]
        plsc.store_scatter(tok_s, [idx], tok)
        plsc.store_scatter(ws_s, [idx], wv[pl.ds(base, 16)])

    base = wid * ROW_W
    nch = ROW_W // CH
    bufs = (rows_v, rows2_v)
    sems = (sem, sem2)
    descs = [None, None]
    descs[0] = pltpu.async_copy(x_hbm.at[tok_s.at[pl.ds(base, CH)]],
                                bufs[0], sems[0])
    for c in range(nch):
        if c + 1 < nch:
            cb = base + (c + 1) * CH
            descs[(c + 1) % 2] = pltpu.async_copy(
                x_hbm.at[tok_s.at[pl.ds(cb, CH)]],
                bufs[(c + 1) % 2], sems[(c + 1) % 2])
        descs[c % 2].wait()
        pltpu.sync_copy(bufs[c % 2], xg_hbm.at[pl.ds(base + c * CH, CH)])
    pltpu.sync_copy(ws_s.at[pl.ds(base, ROW_W)],
                    ws_hbm.at[pl.ds(base, ROW_W)])


def _route_gather(posf, wf, x):
    mesh = plsc.VectorSubcoreMesh(core_axis_name="c", subcore_axis_name="s")
    return pl.kernel(
        _route_gather_body,
        out_type=(
            jax.ShapeDtypeStruct((PADTOT, D), jnp.float32),
            jax.ShapeDtypeStruct((PADTOT,), jnp.float32),
        ),
        mesh=mesh,
        compiler_params=pltpu.CompilerParams(needs_layout_passes=False),
        scratch_types=[
            pltpu.VMEM((NPAIR,), jnp.int32),
            pltpu.VMEM((NPAIR,), jnp.float32),
            pltpu.VMEM((PADTOT,), jnp.int32),
            pltpu.VMEM((PADTOT,), jnp.float32),
            pltpu.VMEM((CH, D), jnp.float32),
            pltpu.VMEM((CH, D), jnp.float32),
            pltpu.SemaphoreType.DMA,
            pltpu.SemaphoreType.DMA,
        ],
    )(posf, wf, x)


def _expert_body(te_ref, tv_ref, xg_ref, ws_ref,
                 w1_ref, b1_ref, g1_ref, be1_ref, w2_ref, b2_ref, eo_ref):
    t = pl.program_id(0)

    @pl.when(tv_ref[t] == 1)
    def _():
        xb = xg_ref[...].astype(jnp.bfloat16)
        h = jnp.dot(xb, w1_ref[0].astype(jnp.bfloat16),
                    preferred_element_type=jnp.float32) + b1_ref[0]
        mu = jnp.mean(h, axis=-1, keepdims=True)
        var = jnp.mean((h - mu) * (h - mu), axis=-1, keepdims=True)
        hn = (h - mu) * lax.rsqrt(var + EPS) * g1_ref[0] + be1_ref[0]
        ha = hn * 0.5 * (1.0 + lax.erf(hn * _INV_SQRT2))
        eo = jnp.dot(ha.astype(jnp.bfloat16), w2_ref[0].astype(jnp.bfloat16),
                     preferred_element_type=jnp.float32) + b2_ref[0]
        wsc = ws_ref[...]
        eo_ref[...] = jnp.where(wsc > 0.0, eo * wsc, 0.0)


def _expert_ffn(ten, tvn, xg, ws2, W1, b1r, g1r, be1r, W2, b2r):
    grid_spec = pltpu.PrefetchScalarGridSpec(
        num_scalar_prefetch=2,
        grid=(NT,),
        in_specs=[
            pl.BlockSpec((T, D), lambda t, te, tv: (t, 0)),
            pl.BlockSpec((T, 1), lambda t, te, tv: (t, 0)),
            pl.BlockSpec((1, D, H), lambda t, te, tv: (te[t], 0, 0)),
            pl.BlockSpec((1, 1, H), lambda t, te, tv: (te[t], 0, 0)),
            pl.BlockSpec((1, 1, H), lambda t, te, tv: (te[t], 0, 0)),
            pl.BlockSpec((1, 1, H), lambda t, te, tv: (te[t], 0, 0)),
            pl.BlockSpec((1, H, H), lambda t, te, tv: (te[t], 0, 0)),
            pl.BlockSpec((1, 1, H), lambda t, te, tv: (te[t], 0, 0)),
        ],
        out_specs=pl.BlockSpec((T, H), lambda t, te, tv: (t, 0)),
    )
    return pl.pallas_call(
        _expert_body,
        grid_spec=grid_spec,
        out_shape=jax.ShapeDtypeStruct((PADTOT, H), jnp.float32),
    )(ten, tvn, xg, ws2, W1, b1r, g1r, be1r, W2, b2r)


def _combine_body(eo_hbm, pos_hbm, out_hbm,
                  p0_v, p1_v, rows0_v, rows1_v, acc_v, sem, sem2):
    wid = lax.axis_index("s") * NC + lax.axis_index("c")
    tokbase = wid * TOK_W
    for sc in range(TOK_W // TSUB):
        tb = tokbase + sc * TSUB
        pltpu.sync_copy(pos_hbm.at[pl.ds(tb, TSUB)], p0_v)
        pltpu.sync_copy(pos_hbm.at[pl.ds(N + tb, TSUB)], p1_v)
        d0 = pltpu.async_copy(eo_hbm.at[p0_v], rows0_v, sem)
        d1 = pltpu.async_copy(eo_hbm.at[p1_v], rows1_v, sem2)
        d0.wait()
        d1.wait()

        @plsc.parallel_loop(0, TSUB * (H // 16), unroll=8)
        def _add(i):
            t = i // (H // 16)
            c = i % (H // 16)
            sl = pl.ds(c * 16, 16)
            acc_v[t, sl] = rows0_v[t, sl] + rows1_v[t, sl]
        pltpu.sync_copy(acc_v, out_hbm.at[pl.ds(tb, TSUB)])


def _combine(eoF, posf):
    mesh = plsc.VectorSubcoreMesh(core_axis_name="c", subcore_axis_name="s")
    return pl.kernel(
        _combine_body,
        out_type=jax.ShapeDtypeStruct((N, H), jnp.float32),
        mesh=mesh,
        compiler_params=pltpu.CompilerParams(needs_layout_passes=False),
        scratch_types=[
            pltpu.VMEM((TSUB,), jnp.int32),
            pltpu.VMEM((TSUB,), jnp.int32),
            pltpu.VMEM((TSUB, H), jnp.float32),
            pltpu.VMEM((TSUB, H), jnp.float32),
            pltpu.VMEM((TSUB, H), jnp.float32),
            pltpu.SemaphoreType.DMA,
            pltpu.SemaphoreType.DMA,
        ],
    )(eoF, posf)


@jax.jit
def kernel(x, W1, b1, g1, be1, W2, b2, Wg, bg):
    bg2 = bg.reshape(1, E)
    b1r = b1.reshape(E, 1, H)
    g1r = g1.reshape(E, 1, H)
    be1r = be1.reshape(E, 1, H)
    b2r = b2.reshape(E, 1, H)

    pos0, pos1, w0, w1, te, tv = _gate_route(x, Wg, bg2)
    posf = jnp.concatenate([pos0.reshape(N), pos1.reshape(N)])
    wf = jnp.concatenate([w0.reshape(N), w1.reshape(N)])
    ten = te.reshape(NT)
    tvn = tv.reshape(NT)

    xg, ws = _route_gather(posf, wf, x)
    eoF = _expert_ffn(ten, tvn, xg, ws.reshape(PADTOT, 1),
                      W1, b1r, g1r, be1r, W2, b2r)
    out = _combine(eoF, posf)
    return out
